# baseline (speedup 1.0000x reference)
import numpy as np
import jax
import jax.numpy as jnp
from jax import lax
from jax.experimental import pallas as pl
from jax.experimental.pallas import tpu as pltpu
from jax.experimental.pallas import tpu_sc as plsc

_INTERP = False

_C = 96
_H = 14
_W = 14
_L = _H * _W
_E = 3
_WIN = 2 * _E + 1
_K2 = _WIN * _WIN
_TOPK = 8
_PAD = 48
_B = 2
_BL = _B * _L
_CH = 16
_NT = 13
_LPAD = _NT * _CH
_FPB = 296
_ROWS = 112

_OFFS = []
_VALID_NP = np.zeros((_L, 64), dtype=np.float32)
for _w in range(_K2):
    _OFFS.append((_w // _WIN - _E) * _W + (_w % _WIN - _E))
for _l in range(_L):
    _i, _j = _l // _W, _l % _W
    for _w in range(_K2):
        _dh, _dw = _w // _WIN - _E, _w % _WIN - _E
        if 0 <= _i + _dh < _H and 0 <= _j + _dw < _W:
            _VALID_NP[_l, _w] = 1.0


def _ka_body(a_ref, w_ref, b_ref, g_ref, bt_ref, xs_ref, xt_ref):
    y = jax.lax.dot_general(a_ref[...], w_ref[...], (((1,), (1,)), ((), ())),
                            preferred_element_type=jnp.float32)
    y = y + b_ref[...]
    mean = jnp.mean(y, axis=0, keepdims=True)
    var = jnp.mean((y - mean) ** 2, axis=0, keepdims=True)
    y = (y - mean) * jax.lax.rsqrt(var + 1e-5) * g_ref[...] + bt_ref[...]
    y = jnp.maximum(y, 0.0)
    xs_ref[...] = y
    for b in range(_B):
        xt_ref[b] = y[b * _L:(b + 1) * _L, :].T


def _ks3_body(br_ref, valid_ref, idx_ref):
    iota = jax.lax.broadcasted_iota(jnp.int32, (_L, 64), 1)
    base = _PAD + jax.lax.broadcasted_iota(jnp.int32, (_L, 1), 0) % _CH
    zpad = jnp.zeros((_PAD, _C), jnp.float32)
    for b in range(_B):
        brp = jnp.concatenate([zpad, br_ref[b], zpad], axis=0)
        center = brp[_PAD:_PAD + _L, :]
        cols = []
        for w in range(_K2):
            off = _OFFS[w]
            nb = brp[_PAD + off:_PAD + off + _L, :]
            d2 = jnp.sum((center - nb) ** 2, axis=1, keepdims=True)
            cols.append(d2)
        d2m = jnp.concatenate(cols, axis=1)
        sim = 1.0 / (1.0 + jnp.sqrt(d2m))
        sim = jnp.concatenate(
            [sim, jnp.full((_L, 64 - _K2), 1e20, jnp.float32)], axis=1)
        vals = jnp.where(valid_ref[...] > 0.0, sim, 1e20)
        icols = []
        for _ in range(_TOPK):
            mn = jnp.min(vals, axis=1, keepdims=True)
            eq = vals <= mn
            idx = jnp.min(jnp.where(eq, iota, 64), axis=1, keepdims=True)
            vals = jnp.where(iota == idx, jnp.float32(3e38), vals)
            off_sel = (idx // _WIN - _E) * _W + (idx % _WIN - _E)
            icols.append(base + off_sel)
        icols.append(jnp.broadcast_to(base, (_L, 8)))
        idx_ref[b] = jnp.concatenate(
            [jnp.concatenate(icols, axis=1),
             jnp.full((_LPAD - _L, 16), _PAD, jnp.int32)], axis=0)


def _sc_gather_body(idx_hbm, fp_hbm, out_hbm, idx_v, rows_v, out_v, sem1,
                    sem2):
    b = lax.axis_index("c")
    t = lax.axis_index("s")

    @pl.when(t < _NT)
    def _():
        h1 = pltpu.async_copy(fp_hbm.at[pl.ds(b * _FPB + t * _CH, _ROWS)],
                              rows_v, sem1)
        h2 = pltpu.async_copy(idx_hbm.at[b, pl.ds(t * _CH, _CH)], idx_v, sem2)
        h2.wait()
        h1.wait()

        def loc(i, carry):
            kinv_v = jnp.full((16,), 1.0 / _TOPK, jnp.float32)
            ivec = idx_v[i, pl.ds(0, 16)]
            acc = [jnp.zeros((16,), jnp.float32) for _ in range(6)]
            for k in range(_TOPK):
                nk = ivec[k]
                for c in range(6):
                    acc[c] = acc[c] + rows_v[nk, pl.ds(16 * c, 16)]
            for c in range(6):
                out_v[i, pl.ds(16 * c, 16)] = (
                    acc[c] * kinv_v - rows_v[_PAD + i, pl.ds(16 * c, 16)])
            return carry

        lax.fori_loop(0, _CH, loc, 0)
        pltpu.sync_copy(out_v, out_hbm.at[b, pl.ds(t * _CH, _CH)])


_SC_CACHE = []


def _sc_gather(idxs, fp):
    if not _SC_CACHE:
        _SC_CACHE.append(pl.kernel(
            _sc_gather_body,
            out_type=jax.ShapeDtypeStruct((_B, _LPAD, _C), jnp.float32),
            mesh=plsc.VectorSubcoreMesh(core_axis_name="c",
                                        subcore_axis_name="s"),
            scratch_types=[
                pltpu.VMEM((_CH, 16), jnp.int32),
                pltpu.VMEM((_ROWS, _C), jnp.float32),
                pltpu.VMEM((_CH, _C), jnp.float32),
                pltpu.SemaphoreType.DMA,
                pltpu.SemaphoreType.DMA,
            ],
        ))
    return _SC_CACHE[0](idxs, fp)


def _kb_body(xs_ref, or_ref, w_ref, b_ref, g_ref, bt_ref, wa_ref, ga_ref,
             bta_ref, rt_ref, o_ref):
    r = jax.lax.broadcasted_iota(jnp.int32, (_L, 1), 0)
    j = r % _W
    mL = (j >= 1).astype(jnp.float32)
    mR = (j <= _W - 2).astype(jnp.float32)
    zeros16 = jnp.zeros((16, 2 * _C), jnp.float32)
    feats = []
    for b in range(_B):
        cat = jnp.concatenate(
            [xs_ref[b * _L:(b + 1) * _L, :], or_ref[b, :_L, :]], axis=1)
        catp = jnp.concatenate([zeros16, cat, zeros16], axis=0)
        acc = jnp.zeros((_L, _C), jnp.float32)
        for dy in range(3):
            for dx in range(3):
                off = (dy - 1) * _W + (dx - 1)
                sh = catp[16 + off:16 + off + _L, :]
                ws = w_ref[(dy * 3 + dx) * 2 * _C:(dy * 3 + dx + 1) * 2 * _C, :]
                part = jnp.dot(sh, ws, preferred_element_type=jnp.float32)
                if dx == 0:
                    part = part * mL
                elif dx == 2:
                    part = part * mR
                acc = acc + part
        feats.append(acc + b_ref[...])
    mean = (jnp.sum(feats[0], axis=0, keepdims=True)
            + jnp.sum(feats[1], axis=0, keepdims=True)) / jnp.float32(_BL)
    var = (jnp.sum((feats[0] - mean) ** 2, axis=0, keepdims=True)
           + jnp.sum((feats[1] - mean) ** 2, axis=0, keepdims=True)) / jnp.float32(_BL)
    scale = jax.lax.rsqrt(var + 1e-5)
    f0 = jnp.maximum((feats[0] - mean) * scale * g_ref[...] + bt_ref[...], 0.0)
    f1 = jnp.maximum((feats[1] - mean) * scale * g_ref[...] + bt_ref[...], 0.0)
    att = jnp.concatenate([jnp.mean(f0, axis=0, keepdims=True),
                           jnp.mean(f1, axis=0, keepdims=True)], axis=0)
    att = jax.lax.dot_general(att, wa_ref[...], (((1,), (1,)), ((), ())),
                              preferred_element_type=jnp.float32)
    m2 = jnp.mean(att, axis=0, keepdims=True)
    v2 = jnp.mean((att - m2) ** 2, axis=0, keepdims=True)
    att = (att - m2) * jax.lax.rsqrt(v2 + 1e-5) * ga_ref[...] + bta_ref[...]
    att = jax.nn.sigmoid(att)
    x = jnp.concatenate([(f0 * att[0:1]).T, (f1 * att[1:2]).T], axis=0)
    o_ref[...] = jnp.dot(x, rt_ref[...], preferred_element_type=jnp.float32)


def _call(body, out_shape, *args):
    if isinstance(out_shape, list):
        os = [jax.ShapeDtypeStruct(s, jnp.float32) for s in out_shape]
    else:
        os = jax.ShapeDtypeStruct(out_shape, jnp.float32)
    return pl.pallas_call(body, out_shape=os, interpret=_INTERP)(*args)


def kernel(x, Wd, bd, gd, betad, Wc, bc, gc, betac, Wa, ga, betaa):
    B, C, H, W, L = _B, _C, _H, _W, _L
    a = x.reshape(B, C, H, 4, W, 4).transpose(0, 2, 4, 1, 3, 5)
    a = a.reshape(B * L, C * 16)
    wd2 = Wd.reshape(C, C * 16)
    valid = jnp.asarray(_VALID_NP)
    xs_rows, x1t = _call(_ka_body, [(_BL, C), (B, C, L)], a, wd2,
                         bd.reshape(1, C), gd.reshape(1, C),
                         betad.reshape(1, C))
    br = x1t.reshape(B, L, C)
    idxs = pl.pallas_call(
        _ks3_body,
        out_shape=jax.ShapeDtypeStruct((B, _LPAD, 16), jnp.int32),
        interpret=_INTERP)(br, valid)
    fp = jnp.pad(xs_rows.reshape(B, L, C),
                 ((0, 0), (_PAD, _FPB - L - _PAD), (0, 0)))
    fp = jnp.pad(fp.reshape(B * _FPB, C), ((0, 48), (0, 0)))
    out_sc = _sc_gather(idxs, fp)
    wc2 = Wc.transpose(2, 3, 1, 0).reshape(9 * 2 * C, C)
    wa2 = Wa.reshape(C, C)
    mh = jax.image.resize(jnp.eye(H, dtype=jnp.float32), (H * 4, H),
                          method="bilinear")
    rt = jnp.kron(mh, mh).T
    y = _call(_kb_body, (B * C, L * 16), xs_rows, out_sc, wc2,
              bc.reshape(1, C), gc.reshape(1, C), betac.reshape(1, C), wa2,
              ga.reshape(1, C), betaa.reshape(1, C), rt)
    return y.reshape(B, C, H * 4, W * 4)
